# baseline (speedup 1.0000x reference)
import functools

import jax
import jax.numpy as jnp
from jax import lax
from jax.experimental import pallas as pl
from jax.experimental.pallas import tpu as pltpu
from jax.experimental.pallas import tpu_sc as plsc

_DIM = 4096
_NE = 64
_NTOK = 32768
_TB = 512

_NW = 32
_RW = _NTOK // _NW
_RG = 16


def _softmax_body(x_ref, w_ref, b_ref, probs_ref):
    x = x_ref[...]
    w = w_ref[...]
    logits = lax.dot_general(x, w, (((1,), (1,)), ((), ())))
    logits = logits + b_ref[...]
    m = jnp.max(logits, axis=1, keepdims=True)
    e = jnp.exp(logits - m)
    s = jnp.sum(e, axis=1, keepdims=True)
    probs_ref[...] = e / s


def _tc_softmax(x, W, b):
    ntok = x.shape[0]
    return pl.pallas_call(
        _softmax_body,
        grid=(ntok // _TB,),
        in_specs=[
            pl.BlockSpec((_TB, _DIM), lambda i: (i, 0)),
            pl.BlockSpec((_NE, _DIM), lambda i: (0, 0)),
            pl.BlockSpec((1, _NE), lambda i: (0, 0)),
        ],
        out_specs=pl.BlockSpec((_TB, _NE), lambda i: (i, 0)),
        out_shape=jax.ShapeDtypeStruct((ntok, _NE), jnp.float32),
    )(x, W, b.reshape(1, _NE))


@functools.partial(
    pl.kernel,
    mesh=plsc.VectorSubcoreMesh(core_axis_name="c", subcore_axis_name="s"),
    compiler_params=pltpu.CompilerParams(needs_layout_passes=False),
    out_type=[
        jax.ShapeDtypeStruct((_NTOK,), jnp.float32),
        jax.ShapeDtypeStruct((_NTOK,), jnp.int32),
    ],
    scratch_types=[
        pltpu.VMEM((_RW * _NE,), jnp.float32),
        pltpu.VMEM((_RW,), jnp.float32),
        pltpu.VMEM((_RW,), jnp.int32),
    ],
)
def _sc_top1(probs_hbm, w_hbm, i_hbm, p_v, w_v, i_v):
    wid = lax.axis_index("s") * 2 + lax.axis_index("c")
    base = wid * _RW
    pltpu.sync_copy(probs_hbm.at[pl.ds(base * _NE, _RW * _NE)], p_v)

    lane_off = lax.iota(jnp.int32, _RG) * _NE

    def group(g, carry):
        r0 = g * _RG
        flat0 = r0 * _NE
        best = jnp.full((_RG,), -1.0, jnp.float32)
        best_i = jnp.zeros((_RG,), jnp.int32)
        for e in range(_NE):
            v = plsc.load_gather(p_v, [lane_off + (flat0 + e)])
            upd = v > best
            best = jnp.where(upd, v, best)
            best_i = jnp.where(upd, e, best_i)
        w_v[pl.ds(r0, _RG)] = best
        i_v[pl.ds(r0, _RG)] = best_i
        return carry

    lax.fori_loop(0, _RW // _RG, group, 0)
    pltpu.sync_copy(w_v, w_hbm.at[pl.ds(base, _RW)])
    pltpu.sync_copy(i_v, i_hbm.at[pl.ds(base, _RW)])


def kernel(x, W, b):
    probs = _tc_softmax(x, W, b)
    wts, idx = _sc_top1(probs.reshape(-1))
    return (wts.reshape(-1, 1), idx.reshape(-1, 1), probs)


# speedup vs baseline: 1.0105x; 1.0105x over previous
import functools

import jax
import jax.numpy as jnp
from jax import lax
from jax.experimental import pallas as pl
from jax.experimental.pallas import tpu as pltpu
from jax.experimental.pallas import tpu_sc as plsc

_DIM = 4096
_NE = 64
_NTOK = 32768
_TB = 1024

_NW = 32
_RW = _NTOK // _NW
_RG = 16


def _softmax_body(x_ref, w_ref, b_ref, probs_ref):
    x = x_ref[...]
    w = w_ref[...]
    logits = lax.dot_general(x, w, (((1,), (1,)), ((), ())))
    logits = logits + b_ref[...]
    m = jnp.max(logits, axis=1, keepdims=True)
    e = jnp.exp(logits - m)
    s = jnp.sum(e, axis=1, keepdims=True)
    probs_ref[...] = e / s


def _tc_softmax(x, W, b):
    ntok = x.shape[0]
    return pl.pallas_call(
        _softmax_body,
        grid=(ntok // _TB,),
        in_specs=[
            pl.BlockSpec((_TB, _DIM), lambda i: (i, 0)),
            pl.BlockSpec((_NE, _DIM), lambda i: (0, 0)),
            pl.BlockSpec((1, _NE), lambda i: (0, 0)),
        ],
        out_specs=pl.BlockSpec((_TB, _NE), lambda i: (i, 0)),
        out_shape=jax.ShapeDtypeStruct((ntok, _NE), jnp.float32),
    )(x, W, b.reshape(1, _NE))


@functools.partial(
    pl.kernel,
    mesh=plsc.VectorSubcoreMesh(core_axis_name="c", subcore_axis_name="s"),
    compiler_params=pltpu.CompilerParams(needs_layout_passes=False),
    out_type=[
        jax.ShapeDtypeStruct((_NTOK,), jnp.float32),
        jax.ShapeDtypeStruct((_NTOK,), jnp.int32),
    ],
    scratch_types=[
        pltpu.VMEM((_RW * _NE,), jnp.float32),
        pltpu.VMEM((_RW,), jnp.float32),
        pltpu.VMEM((_RW,), jnp.int32),
    ],
)
def _sc_top1(probs_hbm, w_hbm, i_hbm, p_v, w_v, i_v):
    wid = lax.axis_index("s") * 2 + lax.axis_index("c")
    base = wid * _RW
    pltpu.sync_copy(probs_hbm.at[pl.ds(base * _NE, _RW * _NE)], p_v)

    lane_off = lax.iota(jnp.int32, _RG) * _NE

    def group(g, carry):
        r0 = g * _RG
        flat0 = r0 * _NE
        best = jnp.full((_RG,), -1.0, jnp.float32)
        best_i = jnp.zeros((_RG,), jnp.int32)
        for e in range(_NE):
            v = plsc.load_gather(p_v, [lane_off + (flat0 + e)])
            upd = v > best
            best = jnp.where(upd, v, best)
            best_i = jnp.where(upd, e, best_i)
        w_v[pl.ds(r0, _RG)] = best
        i_v[pl.ds(r0, _RG)] = best_i
        return carry

    lax.fori_loop(0, _RW // _RG, group, 0)
    pltpu.sync_copy(w_v, w_hbm.at[pl.ds(base, _RW)])
    pltpu.sync_copy(i_v, i_hbm.at[pl.ds(base, _RW)])


def kernel(x, W, b):
    probs = _tc_softmax(x, W, b)
    wts, idx = _sc_top1(probs.reshape(-1))
    return (wts.reshape(-1, 1), idx.reshape(-1, 1), probs)
